# baseline (speedup 1.0000x reference)
import functools

import jax
import jax.numpy as jnp
from jax import lax
from jax.experimental import pallas as pl
from jax.experimental.pallas import tpu as pltpu
from jax.experimental.pallas import tpu_sc as plsc

N = 4194304
BINS = 100
PBINS = 128
NC = 2
NS = 16
L = 16
NW = NC * NS
PER_W = N // NW
CHUNK = 2048
NCHUNK = PER_W // CHUNK
CSTRIDE = 129
HSIZE = 15 * CSTRIDE + PBINS + 1
UNROLL = 1
NSTREAM = CHUNK


def _sc_hist_body(x_hbm, y_hbm, hx_out, hy_out,
                  bufx0, bufx1, bufy0, bufy1, idxb0, idxb1, onesb, hxv, hyv,
                  zrow, yh,
                  semx0, semx1, semy0, semy1, semst0, semst1):
    cid = lax.axis_index("c")
    sid = lax.axis_index("s")
    wid = sid * NC + cid
    base = wid * PER_W

    zeros = jnp.zeros((L,), jnp.float32)
    ones = jnp.ones((L,), jnp.float32)
    lane_f = (lax.iota(jnp.int32, L) * CSTRIDE).astype(jnp.float32)
    ybase = sid * PBINS
    syf = (ybase).astype(jnp.float32)

    def zero_body(i, carry):
        hxv[pl.ds(i * L, L)] = zeros
        hyv[pl.ds(i * L, L)] = zeros
        return carry

    lax.fori_loop(0, HSIZE // L, zero_body, 0)
    for j in range(PBINS // L):
        zrow[pl.ds(j * L, L)] = zeros
    pltpu.sync_copy(zrow, yh.at[pl.ds(ybase, PBINS)])

    def ones_body(i, carry):
        onesb[pl.ds(i * L, L)] = ones
        return carry

    lax.fori_loop(0, CHUNK // L, ones_body, 0)

    bufs = ((bufx0, bufy0, idxb0, semx0, semy0, semst0),
            (bufx1, bufy1, idxb1, semx1, semy1, semst1))

    def descs(c):
        bx, by, _, sx, sy, _ = bufs[c % 2]
        o = base + c * CHUNK
        return (pltpu.make_async_copy(x_hbm.at[pl.ds(o, CHUNK)], bx, sx),
                pltpu.make_async_copy(y_hbm.at[pl.ds(o, CHUNK)], by, sy))

    def stream_desc(c):
        _, _, ib, _, _, st = bufs[c % 2]
        return pltpu.make_async_copy(onesb.at[pl.ds(0, NSTREAM)], yh.at[ib], st)

    def process(c):
        bx, by, ib = bufs[c % 2][0], bufs[c % 2][1], bufs[c % 2][2]

        def vec_body(i, carry2):
            for u in range(UNROLL):
                o = (i * UNROLL + u) * L
                fx = bx[pl.ds(o, L)] * 100.0 + lane_f
                plsc.addupdate_scatter(hxv, [fx.astype(jnp.int32)], ones)
                fy = by[pl.ds(o, L)] * 100.0 + syf
                ib[pl.ds(o, L)] = fy.astype(jnp.int32)
            return carry2

        lax.fori_loop(0, CHUNK // (L * UNROLL), vec_body, 0)

    for d in descs(0):
        d.start()
    for c in range(NCHUNK):
        if c + 1 < NCHUNK:
            for d in descs(c + 1):
                d.start()
        for d in descs(c):
            d.wait()
        if c >= 2:
            stream_desc(c - 2).wait()
        process(c)
        stream_desc(c).start(add=True)

    stream_desc(NCHUNK - 2).wait()
    stream_desc(NCHUNK - 1).wait()

    for step in (8, 4, 2, 1):
        for c in range(step):
            dst = c * CSTRIDE
            src = (c + step) * CSTRIDE
            for j in range(PBINS // L):
                o = j * L
                hxv[pl.ds(dst + o, L)] = hxv[pl.ds(dst + o, L)] + hxv[pl.ds(src + o, L)]
                hyv[pl.ds(dst + o, L)] = hyv[pl.ds(dst + o, L)] + hyv[pl.ds(src + o, L)]

    pltpu.sync_copy(yh.at[pl.ds(ybase, PBINS)], zrow)
    for j in range(PBINS // L):
        o = j * L
        hyv[pl.ds(o, L)] = hyv[pl.ds(o, L)] + zrow[pl.ds(o, L)]

    pltpu.sync_copy(hxv.at[pl.ds(0, PBINS)], hx_out.at[wid])
    pltpu.sync_copy(hyv.at[pl.ds(0, PBINS)], hy_out.at[wid])


_sc_hist = functools.partial(
    pl.kernel,
    out_type=(
        jax.ShapeDtypeStruct((NW, PBINS), jnp.float32),
        jax.ShapeDtypeStruct((NW, PBINS), jnp.float32),
    ),
    mesh=plsc.VectorSubcoreMesh(
        core_axis_name="c", subcore_axis_name="s", num_cores=NC, num_subcores=NS
    ),
    scratch_types=(
        pltpu.VMEM((CHUNK,), jnp.float32),
        pltpu.VMEM((CHUNK,), jnp.float32),
        pltpu.VMEM((CHUNK,), jnp.float32),
        pltpu.VMEM((CHUNK,), jnp.float32),
        pltpu.VMEM((NSTREAM,), jnp.int32),
        pltpu.VMEM((NSTREAM,), jnp.int32),
        pltpu.VMEM((CHUNK,), jnp.float32),
        pltpu.VMEM((HSIZE,), jnp.float32),
        pltpu.VMEM((HSIZE,), jnp.float32),
        pltpu.VMEM((PBINS,), jnp.float32),
        pltpu.VMEM_SHARED((NS * PBINS,), jnp.float32),
        pltpu.SemaphoreType.DMA,
        pltpu.SemaphoreType.DMA,
        pltpu.SemaphoreType.DMA,
        pltpu.SemaphoreType.DMA,
        pltpu.SemaphoreType.DMA,
        pltpu.SemaphoreType.DMA,
    ),
    compiler_params=pltpu.CompilerParams(needs_layout_passes=False),
)(_sc_hist_body)


def _tc_finish_body(hx_ref, hy_ref, eps_ref, out_ref):
    hx = jnp.sum(hx_ref[...], axis=0, keepdims=True)
    hy = jnp.sum(hy_ref[...], axis=0, keepdims=True)
    eps = eps_ref[0]
    joint = hx + hy
    pmi = jnp.log(joint / (hx * hy) + eps)
    valid = lax.broadcasted_iota(jnp.int32, (1, PBINS), 1) < BINS
    pmi = jnp.where(valid, pmi, 0.0)
    mean = jnp.sum(pmi) / BINS
    dev = jnp.where(valid, pmi - mean, 0.0)
    std = jnp.sqrt(jnp.sum(dev * dev) / (BINS - 1))
    out_ref[0, 0] = (mean - std) + (mean + std)


def _tc_finish(hxp, hyp, eps):
    return pl.pallas_call(
        _tc_finish_body,
        out_shape=jax.ShapeDtypeStruct((1, 1), jnp.float32),
        in_specs=[
            pl.BlockSpec(memory_space=pltpu.VMEM),
            pl.BlockSpec(memory_space=pltpu.VMEM),
            pl.BlockSpec(memory_space=pltpu.SMEM),
        ],
        out_specs=pl.BlockSpec(memory_space=pltpu.SMEM),
    )(hxp, hyp, eps)


def kernel(x, y, epsilon):
    hxp, hyp = _sc_hist(x, y)
    eps = jnp.asarray(epsilon, jnp.float32).reshape(1)
    out = _tc_finish(hxp, hyp, eps)
    return out[0, 0]


# speedup vs baseline: 1.0235x; 1.0235x over previous
import functools

import jax
import jax.numpy as jnp
from jax import lax
from jax.experimental import pallas as pl
from jax.experimental.pallas import tpu as pltpu
from jax.experimental.pallas import tpu_sc as plsc

N = 4194304
BINS = 100
PBINS = 128
NC = 2
NS = 16
L = 16
NW = NC * NS
PER_W = N // NW
CHUNK = 4096
NCHUNK = PER_W // CHUNK
CSTRIDE = 129
HSIZE = 15 * CSTRIDE + PBINS + 1
UNROLL = 1
NSTREAM = CHUNK


def _sc_hist_body(x_hbm, y_hbm, hx_out, hy_out,
                  bufx0, bufx1, bufy0, bufy1, idxb0, idxb1, onesb, hxv, hyv,
                  zrow, yh,
                  semx0, semx1, semy0, semy1, semst0, semst1):
    cid = lax.axis_index("c")
    sid = lax.axis_index("s")
    wid = sid * NC + cid
    base = wid * PER_W

    zeros = jnp.zeros((L,), jnp.float32)
    ones = jnp.ones((L,), jnp.float32)
    lane_f = (lax.iota(jnp.int32, L) * CSTRIDE).astype(jnp.float32)
    ybase = sid * PBINS
    syf = (ybase).astype(jnp.float32)

    def zero_body(i, carry):
        hxv[pl.ds(i * L, L)] = zeros
        hyv[pl.ds(i * L, L)] = zeros
        return carry

    lax.fori_loop(0, HSIZE // L, zero_body, 0)
    for j in range(PBINS // L):
        zrow[pl.ds(j * L, L)] = zeros
    pltpu.sync_copy(zrow, yh.at[pl.ds(ybase, PBINS)])

    def ones_body(i, carry):
        onesb[pl.ds(i * L, L)] = ones
        return carry

    lax.fori_loop(0, CHUNK // L, ones_body, 0)

    bufs = ((bufx0, bufy0, idxb0, semx0, semy0, semst0),
            (bufx1, bufy1, idxb1, semx1, semy1, semst1))

    def descs(c):
        bx, by, _, sx, sy, _ = bufs[c % 2]
        o = base + c * CHUNK
        return (pltpu.make_async_copy(x_hbm.at[pl.ds(o, CHUNK)], bx, sx),
                pltpu.make_async_copy(y_hbm.at[pl.ds(o, CHUNK)], by, sy))

    def stream_desc(c):
        _, _, ib, _, _, st = bufs[c % 2]
        return pltpu.make_async_copy(onesb.at[pl.ds(0, NSTREAM)], yh.at[ib], st)

    def process(c):
        bx, by, ib = bufs[c % 2][0], bufs[c % 2][1], bufs[c % 2][2]

        def vec_body(i, carry2):
            for u in range(UNROLL):
                o = (i * UNROLL + u) * L
                fx = bx[pl.ds(o, L)] * 100.0 + lane_f
                plsc.addupdate_scatter(hxv, [fx.astype(jnp.int32)], ones)
                fy = by[pl.ds(o, L)] * 100.0 + syf
                ib[pl.ds(o, L)] = fy.astype(jnp.int32)
            return carry2

        lax.fori_loop(0, CHUNK // (L * UNROLL), vec_body, 0)

    for d in descs(0):
        d.start()
    for c in range(NCHUNK):
        if c + 1 < NCHUNK:
            for d in descs(c + 1):
                d.start()
        for d in descs(c):
            d.wait()
        if c >= 2:
            stream_desc(c - 2).wait()
        process(c)
        stream_desc(c).start(add=True)

    stream_desc(NCHUNK - 2).wait()
    stream_desc(NCHUNK - 1).wait()

    for step in (8, 4, 2, 1):
        for c in range(step):
            dst = c * CSTRIDE
            src = (c + step) * CSTRIDE
            for j in range(PBINS // L):
                o = j * L
                hxv[pl.ds(dst + o, L)] = hxv[pl.ds(dst + o, L)] + hxv[pl.ds(src + o, L)]
                hyv[pl.ds(dst + o, L)] = hyv[pl.ds(dst + o, L)] + hyv[pl.ds(src + o, L)]

    pltpu.sync_copy(yh.at[pl.ds(ybase, PBINS)], zrow)
    for j in range(PBINS // L):
        o = j * L
        hyv[pl.ds(o, L)] = hyv[pl.ds(o, L)] + zrow[pl.ds(o, L)]

    pltpu.sync_copy(hxv.at[pl.ds(0, PBINS)], hx_out.at[wid])
    pltpu.sync_copy(hyv.at[pl.ds(0, PBINS)], hy_out.at[wid])


_sc_hist = functools.partial(
    pl.kernel,
    out_type=(
        jax.ShapeDtypeStruct((NW, PBINS), jnp.float32),
        jax.ShapeDtypeStruct((NW, PBINS), jnp.float32),
    ),
    mesh=plsc.VectorSubcoreMesh(
        core_axis_name="c", subcore_axis_name="s", num_cores=NC, num_subcores=NS
    ),
    scratch_types=(
        pltpu.VMEM((CHUNK,), jnp.float32),
        pltpu.VMEM((CHUNK,), jnp.float32),
        pltpu.VMEM((CHUNK,), jnp.float32),
        pltpu.VMEM((CHUNK,), jnp.float32),
        pltpu.VMEM((NSTREAM,), jnp.int32),
        pltpu.VMEM((NSTREAM,), jnp.int32),
        pltpu.VMEM((CHUNK,), jnp.float32),
        pltpu.VMEM((HSIZE,), jnp.float32),
        pltpu.VMEM((HSIZE,), jnp.float32),
        pltpu.VMEM((PBINS,), jnp.float32),
        pltpu.VMEM_SHARED((NS * PBINS,), jnp.float32),
        pltpu.SemaphoreType.DMA,
        pltpu.SemaphoreType.DMA,
        pltpu.SemaphoreType.DMA,
        pltpu.SemaphoreType.DMA,
        pltpu.SemaphoreType.DMA,
        pltpu.SemaphoreType.DMA,
    ),
    compiler_params=pltpu.CompilerParams(needs_layout_passes=False),
)(_sc_hist_body)


def _tc_finish_body(hx_ref, hy_ref, eps_ref, out_ref):
    hx = jnp.sum(hx_ref[...], axis=0, keepdims=True)
    hy = jnp.sum(hy_ref[...], axis=0, keepdims=True)
    eps = eps_ref[0]
    joint = hx + hy
    pmi = jnp.log(joint / (hx * hy) + eps)
    valid = lax.broadcasted_iota(jnp.int32, (1, PBINS), 1) < BINS
    pmi = jnp.where(valid, pmi, 0.0)
    mean = jnp.sum(pmi) / BINS
    dev = jnp.where(valid, pmi - mean, 0.0)
    std = jnp.sqrt(jnp.sum(dev * dev) / (BINS - 1))
    out_ref[0, 0] = (mean - std) + (mean + std)


def _tc_finish(hxp, hyp, eps):
    return pl.pallas_call(
        _tc_finish_body,
        out_shape=jax.ShapeDtypeStruct((1, 1), jnp.float32),
        in_specs=[
            pl.BlockSpec(memory_space=pltpu.VMEM),
            pl.BlockSpec(memory_space=pltpu.VMEM),
            pl.BlockSpec(memory_space=pltpu.SMEM),
        ],
        out_specs=pl.BlockSpec(memory_space=pltpu.SMEM),
    )(hxp, hyp, eps)


def kernel(x, y, epsilon):
    hxp, hyp = _sc_hist(x, y)
    eps = jnp.asarray(epsilon, jnp.float32).reshape(1)
    out = _tc_finish(hxp, hyp, eps)
    return out[0, 0]
